# baseline (speedup 1.0000x reference)
import functools

import jax
import jax.numpy as jnp
from jax import lax
from jax.experimental import pallas as pl
from jax.experimental.pallas import tpu as pltpu


def _silu(z):
    return z * (1.0 / (1.0 + jnp.exp(-z)))


def _gnn_kernel(
    h_in, x_in,
    wa, wb, wc, be0, we1, be1, wx0, bx0, wx1, wh0a, wh0b, bh0, wh1, bh1,
    wv0, bx1, bv0,
    wemb, bemb, wo0, bo0, wo1, bo1,
    outh, outx, outv,
    hbuf, xbuf, vbuf,
    *, num_nodes, hid, bi, depth,
):
    n = num_nodes
    l = pl.program_id(0)
    b = pl.program_id(1)
    r0 = b * bi
    p = lax.rem(l, 2)
    q = 1 - p

    @pl.when((l == 0) & (b == 0))
    def _init():
        hbuf[0] = jnp.dot(h_in[...], wemb[...],
                          preferred_element_type=jnp.float32) + bemb[...]
        xbuf[0] = x_in[...]
        vbuf[0] = jnp.zeros((n, 3), jnp.float32)

    h_full = hbuf[p]
    h_blk = hbuf[p, pl.ds(r0, bi), :]
    x_full = xbuf[p]
    x_blk = xbuf[p, pl.ds(r0, bi), :]
    v_blk = vbuf[p, pl.ds(r0, bi), :]

    r2b = jnp.sum(x_blk * x_blk, axis=1, keepdims=True)
    r2f = jnp.sum(x_full * x_full, axis=1, keepdims=True)
    u_blk = jnp.concatenate([-2.0 * x_blk,
                             jnp.ones((bi, 1), jnp.float32)], axis=1)
    v_full = jnp.concatenate([x_full, r2f], axis=1)
    d2 = r2b + lax.dot_general(u_blk, v_full, (((1,), (1,)), ((), ())),
                               preferred_element_type=jnp.float32,
                               precision=lax.Precision.HIGHEST)

    d2b = d2.astype(jnp.bfloat16).astype(jnp.float32)
    wcb = wc[l].astype(jnp.bfloat16).astype(jnp.float32)
    a_blk = (jnp.dot(h_blk, wa[l], preferred_element_type=jnp.float32)
             + be0[l])
    b_full = jnp.dot(h_full, wb[l], preferred_element_type=jnp.float32)
    pre = _silu(a_blk[:, None, :] + b_full[None, :, :]
                + d2b[:, :, None] * wcb[None, :, :])

    z = jnp.dot(pre.reshape(bi * n, hid), we1[l],
                preferred_element_type=jnp.float32) + be1[l]
    m = _silu(z).reshape(bi, n, hid)

    ii = lax.broadcasted_iota(jnp.int32, (bi, n), 0) + r0
    jj = lax.broadcasted_iota(jnp.int32, (bi, n), 1)
    adjf = (ii != jj).astype(jnp.float32)
    m = m * adjf[:, :, None]

    m_sum = jnp.sum(m, axis=1)

    t = _silu(jnp.dot(m.reshape(bi * n, hid), wx0[l],
                      preferred_element_type=jnp.float32) + bx0[l])
    tb = t.astype(jnp.bfloat16).astype(jnp.float32)
    wx1b = wx1[l].astype(jnp.bfloat16).astype(jnp.float32)
    coef = (jnp.sum(tb.reshape(bi, n, hid) * wx1b[None, :, :], axis=-1)
            + bx1[l])

    xf_aug = jnp.concatenate([x_full, jnp.ones((n, 1), jnp.float32)],
                             axis=1)
    cagg = jnp.dot(coef, xf_aug, preferred_element_type=jnp.float32,
                   precision=lax.Precision.HIGHEST)
    agg = (x_blk * cagg[:, 3:4] - cagg[:, 0:3]) * (1.0 / (n - 1))

    hb16 = h_blk.astype(jnp.bfloat16).astype(jnp.float32)
    wv0b = wv0[l].astype(jnp.bfloat16).astype(jnp.float32)
    hv = jnp.sum(hb16 * wv0b, axis=1, keepdims=True) + bv0[l]
    v_new = hv * v_blk + agg
    x_new = x_blk + v_new

    u = _silu(jnp.dot(h_blk, wh0a[l], preferred_element_type=jnp.float32)
              + jnp.dot(m_sum, wh0b[l], preferred_element_type=jnp.float32)
              + bh0[l])
    h_new = h_blk + jnp.dot(u, wh1[l],
                            preferred_element_type=jnp.float32) + bh1[l]

    hbuf[q, pl.ds(r0, bi), :] = h_new
    xbuf[q, pl.ds(r0, bi), :] = x_new
    vbuf[q, pl.ds(r0, bi), :] = v_new

    @pl.when(l == depth - 1)
    def _emit():
        g = _silu(jnp.dot(h_new, wo0[...],
                          preferred_element_type=jnp.float32) + bo0[...])
        outh[pl.ds(r0, bi), :] = jnp.dot(
            g, wo1[...], preferred_element_type=jnp.float32) + bo1[...]
        outx[pl.ds(r0, bi), :] = x_new
        outv[pl.ds(r0, bi), :] = v_new


def kernel(h, x, params):
    n, in_f = h.shape
    layers = params["layers"]
    depth = len(layers)
    hid = layers[0]["e1"]["w"].shape[0]
    out_f = params["out1"]["w"].shape[1]
    bi = 256

    def stk(fn):
        return jnp.stack([fn(lp) for lp in layers])

    wa = stk(lambda lp: lp["e0"]["w"][:hid])
    wb = stk(lambda lp: lp["e0"]["w"][hid:2 * hid])
    wc = stk(lambda lp: lp["e0"]["w"][2 * hid][None])
    be0 = stk(lambda lp: lp["e0"]["b"][None])
    we1 = stk(lambda lp: lp["e1"]["w"])
    be1 = stk(lambda lp: lp["e1"]["b"][None])
    wx0 = stk(lambda lp: lp["x0"]["w"])
    bx0 = stk(lambda lp: lp["x0"]["b"][None])
    wx1 = stk(lambda lp: lp["x1"]["w"][:, 0][None])
    wh0a = stk(lambda lp: lp["h0"]["w"][:hid])
    wh0b = stk(lambda lp: lp["h0"]["w"][hid:])
    bh0 = stk(lambda lp: lp["h0"]["b"][None])
    wh1 = stk(lambda lp: lp["h1"]["w"])
    bh1 = stk(lambda lp: lp["h1"]["b"][None])
    wv0 = stk(lambda lp: lp["v0"]["w"][:, 0][None])
    bx1 = stk(lambda lp: lp["x1"]["b"][0])
    bv0 = stk(lambda lp: lp["v0"]["b"][0])

    wemb = params["emb_in"]["w"]
    bemb = params["emb_in"]["b"][None]
    wo0 = params["out0"]["w"]
    bo0 = params["out0"]["b"][None]
    wo1 = params["out1"]["w"]
    bo1 = params["out1"]["b"][None]

    f32 = jnp.float32
    vmem_full = lambda a: pl.BlockSpec(
        a.shape, lambda l, b, nd=a.ndim: (0,) * nd)
    smem_full = pl.BlockSpec(memory_space=pltpu.SMEM)

    args = (h, x, wa, wb, wc, be0, we1, be1, wx0, bx0, wx1,
            wh0a, wh0b, bh0, wh1, bh1, wv0, bx1, bv0,
            wemb, bemb, wo0, bo0, wo1, bo1)
    in_specs = []
    for name, a in zip(
            ("h", "x", "wa", "wb", "wc", "be0", "we1", "be1", "wx0", "bx0",
             "wx1", "wh0a", "wh0b", "bh0", "wh1", "bh1", "wv0", "bx1", "bv0",
             "wemb", "bemb", "wo0", "bo0", "wo1", "bo1"), args):
        in_specs.append(smem_full if name in ("bx1", "bv0") else vmem_full(a))

    out_shape = (
        jax.ShapeDtypeStruct((n, out_f), f32),
        jax.ShapeDtypeStruct((n, 3), f32),
        jax.ShapeDtypeStruct((n, 3), f32),
    )
    out_specs = (
        pl.BlockSpec((n, out_f), lambda l, b: (0, 0)),
        pl.BlockSpec((n, 3), lambda l, b: (0, 0)),
        pl.BlockSpec((n, 3), lambda l, b: (0, 0)),
    )

    body = functools.partial(_gnn_kernel, num_nodes=n, hid=hid, bi=bi,
                             depth=depth)
    return pl.pallas_call(
        body,
        grid=(depth, n // bi),
        in_specs=in_specs,
        out_specs=out_specs,
        out_shape=out_shape,
        scratch_shapes=[
            pltpu.VMEM((2, n, hid), f32),
            pltpu.VMEM((2, n, 3), f32),
            pltpu.VMEM((2, n, 3), f32),
        ],
        compiler_params=pltpu.CompilerParams(
            dimension_semantics=("arbitrary", "arbitrary")),
    )(*args)


# speedup vs baseline: 1.3254x; 1.3254x over previous
import functools

import jax
import jax.numpy as jnp
from jax import lax
from jax.experimental import pallas as pl
from jax.experimental.pallas import tpu as pltpu


def _silu(z):
    return z * (1.0 / (1.0 + jnp.exp(-z)))


def _gnn_kernel(
    h_in, x_in,
    wa, wb, wc, be0, we1, be1, wx0, bx0, wx1, wh0a, wh0b, bh0, wh1, bh1,
    wv0, bx1, bv0,
    wemb, bemb, wo0, bo0, wo1, bo1,
    outh, outx, outv,
    hbuf, xbuf, vbuf,
    *, num_nodes, hid, bi, depth,
):
    n = num_nodes
    l = pl.program_id(0)
    b = pl.program_id(1)
    r0 = b * bi
    p = lax.rem(l, 2)
    q = 1 - p

    @pl.when((l == 0) & (b == 0))
    def _init():
        hbuf[0] = jnp.dot(h_in[...], wemb[...],
                          preferred_element_type=jnp.float32) + bemb[...]
        xbuf[0] = x_in[...]
        vbuf[0] = jnp.zeros((n, 3), jnp.float32)

    h_full = hbuf[p]
    h_blk = hbuf[p, pl.ds(r0, bi), :]
    x_full = xbuf[p]
    x_blk = xbuf[p, pl.ds(r0, bi), :]
    v_blk = vbuf[p, pl.ds(r0, bi), :]

    xt3 = x_full.T
    dx0 = x_blk[:, 0:1] - xt3[0:1, :]
    dx1 = x_blk[:, 1:2] - xt3[1:2, :]
    dx2 = x_blk[:, 2:3] - xt3[2:3, :]
    d2 = (dx0 * dx0 + dx1 * dx1) + dx2 * dx2

    d2b = d2.astype(jnp.bfloat16).astype(jnp.float32)
    wcb = wc[l].astype(jnp.bfloat16).astype(jnp.float32)
    a_blk = (jnp.dot(h_blk, wa[l], preferred_element_type=jnp.float32)
             + be0[l])
    b_full = jnp.dot(h_full, wb[l], preferred_element_type=jnp.float32)
    pre = _silu(a_blk[:, None, :] + b_full[None, :, :]
                + d2b[:, :, None] * wcb[None, :, :])

    z = jnp.dot(pre.reshape(bi * n, hid), we1[l],
                preferred_element_type=jnp.float32) + be1[l]
    m = _silu(z).reshape(bi, n, hid)

    ii = lax.broadcasted_iota(jnp.int32, (bi, n), 0) + r0
    jj = lax.broadcasted_iota(jnp.int32, (bi, n), 1)
    adjf = (ii != jj).astype(jnp.float32)
    m = m * adjf[:, :, None]

    m_sum = jnp.sum(m, axis=1)

    t = _silu(jnp.dot(m.reshape(bi * n, hid), wx0[l],
                      preferred_element_type=jnp.float32) + bx0[l])
    tb = t.astype(jnp.bfloat16).astype(jnp.float32)
    wx1b = wx1[l].astype(jnp.bfloat16).astype(jnp.float32)
    coef = (jnp.sum(tb.reshape(bi, n, hid) * wx1b[None, :, :], axis=-1)
            + bx1[l])

    xf_aug = jnp.concatenate([x_full, jnp.ones((n, 1), jnp.float32)],
                             axis=1)
    cagg = jnp.dot(coef, xf_aug, preferred_element_type=jnp.float32,
                   precision=lax.Precision.HIGHEST)
    agg = (x_blk * cagg[:, 3:4] - cagg[:, 0:3]) * (1.0 / (n - 1))

    hb16 = h_blk.astype(jnp.bfloat16).astype(jnp.float32)
    wv0b = wv0[l].astype(jnp.bfloat16).astype(jnp.float32)
    hv = jnp.sum(hb16 * wv0b, axis=1, keepdims=True) + bv0[l]
    v_new = hv * v_blk + agg
    x_new = x_blk + v_new

    u = _silu(jnp.dot(h_blk, wh0a[l], preferred_element_type=jnp.float32)
              + jnp.dot(m_sum, wh0b[l], preferred_element_type=jnp.float32)
              + bh0[l])
    h_new = h_blk + jnp.dot(u, wh1[l],
                            preferred_element_type=jnp.float32) + bh1[l]

    hbuf[q, pl.ds(r0, bi), :] = h_new
    xbuf[q, pl.ds(r0, bi), :] = x_new
    vbuf[q, pl.ds(r0, bi), :] = v_new

    @pl.when(l == depth - 1)
    def _emit():
        g = _silu(jnp.dot(h_new, wo0[...],
                          preferred_element_type=jnp.float32) + bo0[...])
        outh[pl.ds(r0, bi), :] = jnp.dot(
            g, wo1[...], preferred_element_type=jnp.float32) + bo1[...]
        outx[pl.ds(r0, bi), :] = x_new
        outv[pl.ds(r0, bi), :] = v_new


def kernel(h, x, params):
    n, in_f = h.shape
    layers = params["layers"]
    depth = len(layers)
    hid = layers[0]["e1"]["w"].shape[0]
    out_f = params["out1"]["w"].shape[1]
    bi = 128

    def stk(fn):
        return jnp.stack([fn(lp) for lp in layers])

    wa = stk(lambda lp: lp["e0"]["w"][:hid])
    wb = stk(lambda lp: lp["e0"]["w"][hid:2 * hid])
    wc = stk(lambda lp: lp["e0"]["w"][2 * hid][None])
    be0 = stk(lambda lp: lp["e0"]["b"][None])
    we1 = stk(lambda lp: lp["e1"]["w"])
    be1 = stk(lambda lp: lp["e1"]["b"][None])
    wx0 = stk(lambda lp: lp["x0"]["w"])
    bx0 = stk(lambda lp: lp["x0"]["b"][None])
    wx1 = stk(lambda lp: lp["x1"]["w"][:, 0][None])
    wh0a = stk(lambda lp: lp["h0"]["w"][:hid])
    wh0b = stk(lambda lp: lp["h0"]["w"][hid:])
    bh0 = stk(lambda lp: lp["h0"]["b"][None])
    wh1 = stk(lambda lp: lp["h1"]["w"])
    bh1 = stk(lambda lp: lp["h1"]["b"][None])
    wv0 = stk(lambda lp: lp["v0"]["w"][:, 0][None])
    bx1 = stk(lambda lp: lp["x1"]["b"][0])
    bv0 = stk(lambda lp: lp["v0"]["b"][0])

    wemb = params["emb_in"]["w"]
    bemb = params["emb_in"]["b"][None]
    wo0 = params["out0"]["w"]
    bo0 = params["out0"]["b"][None]
    wo1 = params["out1"]["w"]
    bo1 = params["out1"]["b"][None]

    f32 = jnp.float32
    vmem_full = lambda a: pl.BlockSpec(
        a.shape, lambda l, b, nd=a.ndim: (0,) * nd)
    smem_full = pl.BlockSpec(memory_space=pltpu.SMEM)

    args = (h, x, wa, wb, wc, be0, we1, be1, wx0, bx0, wx1,
            wh0a, wh0b, bh0, wh1, bh1, wv0, bx1, bv0,
            wemb, bemb, wo0, bo0, wo1, bo1)
    in_specs = []
    for name, a in zip(
            ("h", "x", "wa", "wb", "wc", "be0", "we1", "be1", "wx0", "bx0",
             "wx1", "wh0a", "wh0b", "bh0", "wh1", "bh1", "wv0", "bx1", "bv0",
             "wemb", "bemb", "wo0", "bo0", "wo1", "bo1"), args):
        in_specs.append(smem_full if name in ("bx1", "bv0") else vmem_full(a))

    out_shape = (
        jax.ShapeDtypeStruct((n, out_f), f32),
        jax.ShapeDtypeStruct((n, 3), f32),
        jax.ShapeDtypeStruct((n, 3), f32),
    )
    out_specs = (
        pl.BlockSpec((n, out_f), lambda l, b: (0, 0)),
        pl.BlockSpec((n, 3), lambda l, b: (0, 0)),
        pl.BlockSpec((n, 3), lambda l, b: (0, 0)),
    )

    body = functools.partial(_gnn_kernel, num_nodes=n, hid=hid, bi=bi,
                             depth=depth)
    return pl.pallas_call(
        body,
        grid=(depth, n // bi),
        in_specs=in_specs,
        out_specs=out_specs,
        out_shape=out_shape,
        scratch_shapes=[
            pltpu.VMEM((2, n, hid), f32),
            pltpu.VMEM((2, n, 3), f32),
            pltpu.VMEM((2, n, 3), f32),
        ],
        compiler_params=pltpu.CompilerParams(
            dimension_semantics=("arbitrary", "arbitrary")),
    )(*args)


# speedup vs baseline: 1.3331x; 1.0058x over previous
import functools

import jax
import jax.numpy as jnp
from jax import lax
from jax.experimental import pallas as pl
from jax.experimental.pallas import tpu as pltpu


def _silu(z):
    return z * (1.0 / (1.0 + jnp.exp(-z)))


def _gnn_kernel(
    h_in, x_in,
    wa, wb, wc, be0, we1, be1, wx0, bx0, wx1, wh0a, wh0b, bh0, wh1, bh1,
    wv0, bx1, bv0,
    wemb, bemb, wo0, bo0, wo1, bo1,
    outh, outx, outv,
    hbuf, xbuf, vbuf,
    *, num_nodes, hid, bi, depth,
):
    n = num_nodes
    l = pl.program_id(0)
    b = pl.program_id(1)
    r0 = b * bi
    p = lax.rem(l, 2)
    q = 1 - p

    @pl.when((l == 0) & (b == 0))
    def _init():
        hbuf[0] = jnp.dot(h_in[...], wemb[...],
                          preferred_element_type=jnp.float32) + bemb[...]
        xbuf[0] = x_in[...]
        vbuf[0] = jnp.zeros((n, 3), jnp.float32)

    h_full = hbuf[p]
    h_blk = hbuf[p, pl.ds(r0, bi), :]
    x_full = xbuf[p]
    x_blk = xbuf[p, pl.ds(r0, bi), :]
    v_blk = vbuf[p, pl.ds(r0, bi), :]

    xt3 = x_full.T
    dx0 = x_blk[:, 0:1] - xt3[0:1, :]
    dx1 = x_blk[:, 1:2] - xt3[1:2, :]
    dx2 = x_blk[:, 2:3] - xt3[2:3, :]
    d2 = (dx0 * dx0 + dx1 * dx1) + dx2 * dx2

    d2b = d2.astype(jnp.bfloat16).astype(jnp.float32)
    wcb = wc[l].astype(jnp.bfloat16).astype(jnp.float32)
    a_blk = (jnp.dot(h_blk, wa[l], preferred_element_type=jnp.float32)
             + be0[l])
    b_full = jnp.dot(h_full, wb[l], preferred_element_type=jnp.float32)
    pre = _silu(a_blk[:, None, :] + b_full[None, :, :]
                + d2b[:, :, None] * wcb[None, :, :])

    z = jnp.dot(pre.reshape(bi * n, hid), we1[l],
                preferred_element_type=jnp.float32) + be1[l]
    m = _silu(z).reshape(bi, n, hid)

    ii = lax.broadcasted_iota(jnp.int32, (bi, n), 0) + r0
    jj = lax.broadcasted_iota(jnp.int32, (bi, n), 1)
    adjf = (ii != jj).astype(jnp.float32)
    m = m * adjf[:, :, None]

    m_sum = jnp.sum(m, axis=1)

    t = _silu(jnp.dot(m.reshape(bi * n, hid), wx0[l],
                      preferred_element_type=jnp.float32) + bx0[l])
    tb = t.astype(jnp.bfloat16).astype(jnp.float32)
    wx1b = wx1[l].astype(jnp.bfloat16).astype(jnp.float32)
    coef = (jnp.sum(tb.reshape(bi, n, hid) * wx1b[None, :, :], axis=-1)
            + bx1[l])

    scale = 1.0 / (n - 1)
    agg = jnp.concatenate(
        [jnp.sum(dx0 * coef, axis=1, keepdims=True),
         jnp.sum(dx1 * coef, axis=1, keepdims=True),
         jnp.sum(dx2 * coef, axis=1, keepdims=True)], axis=1) * scale

    hb16 = h_blk.astype(jnp.bfloat16).astype(jnp.float32)
    wv0b = wv0[l].astype(jnp.bfloat16).astype(jnp.float32)
    hv = jnp.sum(hb16 * wv0b, axis=1, keepdims=True) + bv0[l]
    v_new = hv * v_blk + agg
    x_new = x_blk + v_new

    u = _silu(jnp.dot(h_blk, wh0a[l], preferred_element_type=jnp.float32)
              + jnp.dot(m_sum, wh0b[l], preferred_element_type=jnp.float32)
              + bh0[l])
    h_new = h_blk + jnp.dot(u, wh1[l],
                            preferred_element_type=jnp.float32) + bh1[l]

    hbuf[q, pl.ds(r0, bi), :] = h_new
    xbuf[q, pl.ds(r0, bi), :] = x_new
    vbuf[q, pl.ds(r0, bi), :] = v_new

    @pl.when(l == depth - 1)
    def _emit():
        g = _silu(jnp.dot(h_new, wo0[...],
                          preferred_element_type=jnp.float32) + bo0[...])
        outh[pl.ds(r0, bi), :] = jnp.dot(
            g, wo1[...], preferred_element_type=jnp.float32) + bo1[...]
        outx[pl.ds(r0, bi), :] = x_new
        outv[pl.ds(r0, bi), :] = v_new


def kernel(h, x, params):
    n, in_f = h.shape
    layers = params["layers"]
    depth = len(layers)
    hid = layers[0]["e1"]["w"].shape[0]
    out_f = params["out1"]["w"].shape[1]
    bi = 128

    def stk(fn):
        return jnp.stack([fn(lp) for lp in layers])

    wa = stk(lambda lp: lp["e0"]["w"][:hid])
    wb = stk(lambda lp: lp["e0"]["w"][hid:2 * hid])
    wc = stk(lambda lp: lp["e0"]["w"][2 * hid][None])
    be0 = stk(lambda lp: lp["e0"]["b"][None])
    we1 = stk(lambda lp: lp["e1"]["w"])
    be1 = stk(lambda lp: lp["e1"]["b"][None])
    wx0 = stk(lambda lp: lp["x0"]["w"])
    bx0 = stk(lambda lp: lp["x0"]["b"][None])
    wx1 = stk(lambda lp: lp["x1"]["w"][:, 0][None])
    wh0a = stk(lambda lp: lp["h0"]["w"][:hid])
    wh0b = stk(lambda lp: lp["h0"]["w"][hid:])
    bh0 = stk(lambda lp: lp["h0"]["b"][None])
    wh1 = stk(lambda lp: lp["h1"]["w"])
    bh1 = stk(lambda lp: lp["h1"]["b"][None])
    wv0 = stk(lambda lp: lp["v0"]["w"][:, 0][None])
    bx1 = stk(lambda lp: lp["x1"]["b"][0])
    bv0 = stk(lambda lp: lp["v0"]["b"][0])

    wemb = params["emb_in"]["w"]
    bemb = params["emb_in"]["b"][None]
    wo0 = params["out0"]["w"]
    bo0 = params["out0"]["b"][None]
    wo1 = params["out1"]["w"]
    bo1 = params["out1"]["b"][None]

    f32 = jnp.float32
    vmem_full = lambda a: pl.BlockSpec(
        a.shape, lambda l, b, nd=a.ndim: (0,) * nd)
    smem_full = pl.BlockSpec(memory_space=pltpu.SMEM)

    args = (h, x, wa, wb, wc, be0, we1, be1, wx0, bx0, wx1,
            wh0a, wh0b, bh0, wh1, bh1, wv0, bx1, bv0,
            wemb, bemb, wo0, bo0, wo1, bo1)
    in_specs = []
    for name, a in zip(
            ("h", "x", "wa", "wb", "wc", "be0", "we1", "be1", "wx0", "bx0",
             "wx1", "wh0a", "wh0b", "bh0", "wh1", "bh1", "wv0", "bx1", "bv0",
             "wemb", "bemb", "wo0", "bo0", "wo1", "bo1"), args):
        in_specs.append(smem_full if name in ("bx1", "bv0") else vmem_full(a))

    out_shape = (
        jax.ShapeDtypeStruct((n, out_f), f32),
        jax.ShapeDtypeStruct((n, 3), f32),
        jax.ShapeDtypeStruct((n, 3), f32),
    )
    out_specs = (
        pl.BlockSpec((n, out_f), lambda l, b: (0, 0)),
        pl.BlockSpec((n, 3), lambda l, b: (0, 0)),
        pl.BlockSpec((n, 3), lambda l, b: (0, 0)),
    )

    body = functools.partial(_gnn_kernel, num_nodes=n, hid=hid, bi=bi,
                             depth=depth)
    return pl.pallas_call(
        body,
        grid=(depth, n // bi),
        in_specs=in_specs,
        out_specs=out_specs,
        out_shape=out_shape,
        scratch_shapes=[
            pltpu.VMEM((2, n, hid), f32),
            pltpu.VMEM((2, n, 3), f32),
            pltpu.VMEM((2, n, 3), f32),
        ],
        compiler_params=pltpu.CompilerParams(
            dimension_semantics=("arbitrary", "arbitrary")),
    )(*args)


# speedup vs baseline: 1.6016x; 1.2014x over previous
import functools

import jax
import jax.numpy as jnp
from jax import lax
from jax.experimental import pallas as pl
from jax.experimental.pallas import tpu as pltpu


def _silu(z):
    h = 0.5 * z
    return h + h * jnp.tanh(h)


def _gnn_kernel(
    h_in, x_in,
    wa, wb, wc, be0, we1, be1, wx0, bx0, wx1, wh0a, wh0b, bh0, wh1, bh1,
    wv0, bx1, bv0,
    wemb, bemb, wo0, bo0, wo1, bo1,
    outh, outx, outv,
    hbuf, xbuf, vbuf,
    *, num_nodes, hid, bi, depth,
):
    n = num_nodes
    l = pl.program_id(0)
    b = pl.program_id(1)
    r0 = b * bi
    p = lax.rem(l, 2)
    q = 1 - p

    @pl.when((l == 0) & (b == 0))
    def _init():
        hbuf[0] = jnp.dot(h_in[...], wemb[...],
                          preferred_element_type=jnp.float32) + bemb[...]
        xbuf[0] = x_in[...]
        vbuf[0] = jnp.zeros((n, 3), jnp.float32)

    h_full = hbuf[p]
    h_blk = hbuf[p, pl.ds(r0, bi), :]
    x_full = xbuf[p]
    x_blk = xbuf[p, pl.ds(r0, bi), :]
    v_blk = vbuf[p, pl.ds(r0, bi), :]

    xt3 = x_full.T
    dx0 = x_blk[:, 0:1] - xt3[0:1, :]
    dx1 = x_blk[:, 1:2] - xt3[1:2, :]
    dx2 = x_blk[:, 2:3] - xt3[2:3, :]
    d2 = (dx0 * dx0 + dx1 * dx1) + dx2 * dx2

    d2b = d2.astype(jnp.bfloat16).astype(jnp.float32)
    wcb = wc[l].astype(jnp.bfloat16).astype(jnp.float32)
    a_blk = (jnp.dot(h_blk, wa[l], preferred_element_type=jnp.float32)
             + be0[l])
    b_full = jnp.dot(h_full, wb[l], preferred_element_type=jnp.float32)
    pre = _silu(a_blk[:, None, :] + b_full[None, :, :]
                + d2b[:, :, None] * wcb[None, :, :])

    z = jnp.dot(pre.reshape(bi * n, hid), we1[l],
                preferred_element_type=jnp.float32) + be1[l]
    m = _silu(z).reshape(bi, n, hid)

    ii = lax.broadcasted_iota(jnp.int32, (bi, n), 0) + r0
    jj = lax.broadcasted_iota(jnp.int32, (bi, n), 1)
    adjf = (ii != jj).astype(jnp.float32)
    m = m * adjf[:, :, None]

    m_sum = jnp.sum(m, axis=1)

    t = _silu(jnp.dot(m.reshape(bi * n, hid), wx0[l],
                      preferred_element_type=jnp.float32) + bx0[l])
    tb = t.astype(jnp.bfloat16).astype(jnp.float32)
    wx1b = wx1[l].astype(jnp.bfloat16).astype(jnp.float32)
    coef = (jnp.sum(tb.reshape(bi, n, hid) * wx1b[None, :, :], axis=-1)
            + bx1[l])

    scale = 1.0 / (n - 1)
    agg = jnp.concatenate(
        [jnp.sum(dx0 * coef, axis=1, keepdims=True),
         jnp.sum(dx1 * coef, axis=1, keepdims=True),
         jnp.sum(dx2 * coef, axis=1, keepdims=True)], axis=1) * scale

    hb16 = h_blk.astype(jnp.bfloat16).astype(jnp.float32)
    wv0b = wv0[l].astype(jnp.bfloat16).astype(jnp.float32)
    hv = jnp.sum(hb16 * wv0b, axis=1, keepdims=True) + bv0[l]
    v_new = hv * v_blk + agg
    x_new = x_blk + v_new

    u = _silu(jnp.dot(h_blk, wh0a[l], preferred_element_type=jnp.float32)
              + jnp.dot(m_sum, wh0b[l], preferred_element_type=jnp.float32)
              + bh0[l])
    h_new = h_blk + jnp.dot(u, wh1[l],
                            preferred_element_type=jnp.float32) + bh1[l]

    hbuf[q, pl.ds(r0, bi), :] = h_new
    xbuf[q, pl.ds(r0, bi), :] = x_new
    vbuf[q, pl.ds(r0, bi), :] = v_new

    @pl.when(l == depth - 1)
    def _emit():
        g = _silu(jnp.dot(h_new, wo0[...],
                          preferred_element_type=jnp.float32) + bo0[...])
        outh[pl.ds(r0, bi), :] = jnp.dot(
            g, wo1[...], preferred_element_type=jnp.float32) + bo1[...]
        outx[pl.ds(r0, bi), :] = x_new
        outv[pl.ds(r0, bi), :] = v_new


def kernel(h, x, params):
    n, in_f = h.shape
    layers = params["layers"]
    depth = len(layers)
    hid = layers[0]["e1"]["w"].shape[0]
    out_f = params["out1"]["w"].shape[1]
    bi = 128

    def stk(fn):
        return jnp.stack([fn(lp) for lp in layers])

    wa = stk(lambda lp: lp["e0"]["w"][:hid])
    wb = stk(lambda lp: lp["e0"]["w"][hid:2 * hid])
    wc = stk(lambda lp: lp["e0"]["w"][2 * hid][None])
    be0 = stk(lambda lp: lp["e0"]["b"][None])
    we1 = stk(lambda lp: lp["e1"]["w"])
    be1 = stk(lambda lp: lp["e1"]["b"][None])
    wx0 = stk(lambda lp: lp["x0"]["w"])
    bx0 = stk(lambda lp: lp["x0"]["b"][None])
    wx1 = stk(lambda lp: lp["x1"]["w"][:, 0][None])
    wh0a = stk(lambda lp: lp["h0"]["w"][:hid])
    wh0b = stk(lambda lp: lp["h0"]["w"][hid:])
    bh0 = stk(lambda lp: lp["h0"]["b"][None])
    wh1 = stk(lambda lp: lp["h1"]["w"])
    bh1 = stk(lambda lp: lp["h1"]["b"][None])
    wv0 = stk(lambda lp: lp["v0"]["w"][:, 0][None])
    bx1 = stk(lambda lp: lp["x1"]["b"][0])
    bv0 = stk(lambda lp: lp["v0"]["b"][0])

    wemb = params["emb_in"]["w"]
    bemb = params["emb_in"]["b"][None]
    wo0 = params["out0"]["w"]
    bo0 = params["out0"]["b"][None]
    wo1 = params["out1"]["w"]
    bo1 = params["out1"]["b"][None]

    f32 = jnp.float32
    vmem_full = lambda a: pl.BlockSpec(
        a.shape, lambda l, b, nd=a.ndim: (0,) * nd)
    smem_full = pl.BlockSpec(memory_space=pltpu.SMEM)

    args = (h, x, wa, wb, wc, be0, we1, be1, wx0, bx0, wx1,
            wh0a, wh0b, bh0, wh1, bh1, wv0, bx1, bv0,
            wemb, bemb, wo0, bo0, wo1, bo1)
    in_specs = []
    for name, a in zip(
            ("h", "x", "wa", "wb", "wc", "be0", "we1", "be1", "wx0", "bx0",
             "wx1", "wh0a", "wh0b", "bh0", "wh1", "bh1", "wv0", "bx1", "bv0",
             "wemb", "bemb", "wo0", "bo0", "wo1", "bo1"), args):
        in_specs.append(smem_full if name in ("bx1", "bv0") else vmem_full(a))

    out_shape = (
        jax.ShapeDtypeStruct((n, out_f), f32),
        jax.ShapeDtypeStruct((n, 3), f32),
        jax.ShapeDtypeStruct((n, 3), f32),
    )
    out_specs = (
        pl.BlockSpec((n, out_f), lambda l, b: (0, 0)),
        pl.BlockSpec((n, 3), lambda l, b: (0, 0)),
        pl.BlockSpec((n, 3), lambda l, b: (0, 0)),
    )

    body = functools.partial(_gnn_kernel, num_nodes=n, hid=hid, bi=bi,
                             depth=depth)
    return pl.pallas_call(
        body,
        grid=(depth, n // bi),
        in_specs=in_specs,
        out_specs=out_specs,
        out_shape=out_shape,
        scratch_shapes=[
            pltpu.VMEM((2, n, hid), f32),
            pltpu.VMEM((2, n, 3), f32),
            pltpu.VMEM((2, n, 3), f32),
        ],
        compiler_params=pltpu.CompilerParams(
            dimension_semantics=("arbitrary", "arbitrary")),
    )(*args)


# speedup vs baseline: 1.7534x; 1.0948x over previous
import functools

import jax
import jax.numpy as jnp
from jax import lax
from jax.experimental import pallas as pl
from jax.experimental.pallas import tpu as pltpu


def _silu(z):
    h = 0.5 * z
    return h + h * jnp.tanh(h)


def _gnn_kernel(
    h_in, x_in,
    wa, wb, wc, be0, we1, be1, wx0, bx0, wx1, wh0a, wh0b, bh0, wh1, bh1,
    wv0, bx1, bv0,
    wemb, bemb, wo0, bo0, wo1, bo1,
    outh, outx, outv,
    hbuf, xbuf, vbuf,
    *, num_nodes, hid, bi, depth,
):
    n = num_nodes
    l = pl.program_id(0)
    b = pl.program_id(1)
    r0 = b * bi
    p = lax.rem(l, 2)
    q = 1 - p

    @pl.when((l == 0) & (b == 0))
    def _init():
        hbuf[0] = jnp.dot(h_in[...], wemb[...],
                          preferred_element_type=jnp.float32) + bemb[...]
        xbuf[0] = x_in[...]
        vbuf[0] = jnp.zeros((n, 3), jnp.float32)

    h_full = hbuf[p]
    h_blk = hbuf[p, pl.ds(r0, bi), :]
    x_full = xbuf[p]
    x_blk = xbuf[p, pl.ds(r0, bi), :]
    v_blk = vbuf[p, pl.ds(r0, bi), :]

    xt3 = x_full.T
    dx0 = x_blk[:, 0:1] - xt3[0:1, :]
    dx1 = x_blk[:, 1:2] - xt3[1:2, :]
    dx2 = x_blk[:, 2:3] - xt3[2:3, :]
    d2 = (dx0 * dx0 + dx1 * dx1) + dx2 * dx2

    d2b = d2.astype(jnp.bfloat16).astype(jnp.float32)
    wcb = wc[l].astype(jnp.bfloat16).astype(jnp.float32)
    a_blk = (jnp.dot(h_blk, wa[l], preferred_element_type=jnp.float32)
             + be0[l])
    b_full = jnp.dot(h_full, wb[l], preferred_element_type=jnp.float32)
    pre = _silu(a_blk[:, None, :] + b_full[None, :, :]
                + d2b[:, :, None] * wcb[None, :, :])

    z = jnp.dot(pre.reshape(bi * n, hid), we1[l],
                preferred_element_type=jnp.float32) + be1[l]
    m = _silu(z).reshape(bi, n, hid)

    b_diag = jnp.dot(h_blk, wb[l], preferred_element_type=jnp.float32)
    pre_diag = _silu(a_blk + b_diag)
    m_diag = _silu(jnp.dot(pre_diag, we1[l],
                           preferred_element_type=jnp.float32) + be1[l])
    m_sum = jnp.sum(m, axis=1) - m_diag

    t = _silu(jnp.dot(m.reshape(bi * n, hid), wx0[l],
                      preferred_element_type=jnp.float32) + bx0[l])
    tb = t.astype(jnp.bfloat16).astype(jnp.float32)
    wx1b = wx1[l].astype(jnp.bfloat16).astype(jnp.float32)
    coef = (jnp.sum(tb.reshape(bi, n, hid) * wx1b[None, :, :], axis=-1)
            + bx1[l])

    scale = 1.0 / (n - 1)
    agg = jnp.concatenate(
        [jnp.sum(dx0 * coef, axis=1, keepdims=True),
         jnp.sum(dx1 * coef, axis=1, keepdims=True),
         jnp.sum(dx2 * coef, axis=1, keepdims=True)], axis=1) * scale

    hb16 = h_blk.astype(jnp.bfloat16).astype(jnp.float32)
    wv0b = wv0[l].astype(jnp.bfloat16).astype(jnp.float32)
    hv = jnp.sum(hb16 * wv0b, axis=1, keepdims=True) + bv0[l]
    v_new = hv * v_blk + agg
    x_new = x_blk + v_new

    u = _silu(jnp.dot(h_blk, wh0a[l], preferred_element_type=jnp.float32)
              + jnp.dot(m_sum, wh0b[l], preferred_element_type=jnp.float32)
              + bh0[l])
    h_new = h_blk + jnp.dot(u, wh1[l],
                            preferred_element_type=jnp.float32) + bh1[l]

    hbuf[q, pl.ds(r0, bi), :] = h_new
    xbuf[q, pl.ds(r0, bi), :] = x_new
    vbuf[q, pl.ds(r0, bi), :] = v_new

    @pl.when(l == depth - 1)
    def _emit():
        g = _silu(jnp.dot(h_new, wo0[...],
                          preferred_element_type=jnp.float32) + bo0[...])
        outh[pl.ds(r0, bi), :] = jnp.dot(
            g, wo1[...], preferred_element_type=jnp.float32) + bo1[...]
        outx[pl.ds(r0, bi), :] = x_new
        outv[pl.ds(r0, bi), :] = v_new


def kernel(h, x, params):
    n, in_f = h.shape
    layers = params["layers"]
    depth = len(layers)
    hid = layers[0]["e1"]["w"].shape[0]
    out_f = params["out1"]["w"].shape[1]
    bi = 128

    def stk(fn):
        return jnp.stack([fn(lp) for lp in layers])

    wa = stk(lambda lp: lp["e0"]["w"][:hid])
    wb = stk(lambda lp: lp["e0"]["w"][hid:2 * hid])
    wc = stk(lambda lp: lp["e0"]["w"][2 * hid][None])
    be0 = stk(lambda lp: lp["e0"]["b"][None])
    we1 = stk(lambda lp: lp["e1"]["w"])
    be1 = stk(lambda lp: lp["e1"]["b"][None])
    wx0 = stk(lambda lp: lp["x0"]["w"])
    bx0 = stk(lambda lp: lp["x0"]["b"][None])
    wx1 = stk(lambda lp: lp["x1"]["w"][:, 0][None])
    wh0a = stk(lambda lp: lp["h0"]["w"][:hid])
    wh0b = stk(lambda lp: lp["h0"]["w"][hid:])
    bh0 = stk(lambda lp: lp["h0"]["b"][None])
    wh1 = stk(lambda lp: lp["h1"]["w"])
    bh1 = stk(lambda lp: lp["h1"]["b"][None])
    wv0 = stk(lambda lp: lp["v0"]["w"][:, 0][None])
    bx1 = stk(lambda lp: lp["x1"]["b"][0])
    bv0 = stk(lambda lp: lp["v0"]["b"][0])

    wemb = params["emb_in"]["w"]
    bemb = params["emb_in"]["b"][None]
    wo0 = params["out0"]["w"]
    bo0 = params["out0"]["b"][None]
    wo1 = params["out1"]["w"]
    bo1 = params["out1"]["b"][None]

    f32 = jnp.float32
    vmem_full = lambda a: pl.BlockSpec(
        a.shape, lambda l, b, nd=a.ndim: (0,) * nd)
    smem_full = pl.BlockSpec(memory_space=pltpu.SMEM)

    args = (h, x, wa, wb, wc, be0, we1, be1, wx0, bx0, wx1,
            wh0a, wh0b, bh0, wh1, bh1, wv0, bx1, bv0,
            wemb, bemb, wo0, bo0, wo1, bo1)
    in_specs = []
    for name, a in zip(
            ("h", "x", "wa", "wb", "wc", "be0", "we1", "be1", "wx0", "bx0",
             "wx1", "wh0a", "wh0b", "bh0", "wh1", "bh1", "wv0", "bx1", "bv0",
             "wemb", "bemb", "wo0", "bo0", "wo1", "bo1"), args):
        in_specs.append(smem_full if name in ("bx1", "bv0") else vmem_full(a))

    out_shape = (
        jax.ShapeDtypeStruct((n, out_f), f32),
        jax.ShapeDtypeStruct((n, 3), f32),
        jax.ShapeDtypeStruct((n, 3), f32),
    )
    out_specs = (
        pl.BlockSpec((n, out_f), lambda l, b: (0, 0)),
        pl.BlockSpec((n, 3), lambda l, b: (0, 0)),
        pl.BlockSpec((n, 3), lambda l, b: (0, 0)),
    )

    body = functools.partial(_gnn_kernel, num_nodes=n, hid=hid, bi=bi,
                             depth=depth)
    return pl.pallas_call(
        body,
        grid=(depth, n // bi),
        in_specs=in_specs,
        out_specs=out_specs,
        out_shape=out_shape,
        scratch_shapes=[
            pltpu.VMEM((2, n, hid), f32),
            pltpu.VMEM((2, n, 3), f32),
            pltpu.VMEM((2, n, 3), f32),
        ],
        compiler_params=pltpu.CompilerParams(
            dimension_semantics=("arbitrary", "arbitrary")),
    )(*args)


# speedup vs baseline: 1.8112x; 1.0329x over previous
import functools

import jax
import jax.numpy as jnp
from jax import lax
from jax.experimental import pallas as pl
from jax.experimental.pallas import tpu as pltpu


def _silu_h(h):
    return h + h * jnp.tanh(h)


def _gnn_kernel(
    h_in, x_in,
    wa, wb, wc, be0, we1, be1, wx0, bx0, wx1, wh0a, wh0b, bh0, wh1, bh1,
    wv0, bx1, bv0,
    wemb, bemb, wo0, bo0, wo1, bo1,
    outh, outx, outv,
    hbuf, xbuf, vbuf,
    *, num_nodes, hid, bi, depth,
):
    n = num_nodes
    l = pl.program_id(0)
    b = pl.program_id(1)
    r0 = b * bi
    p = lax.rem(l, 2)
    q = 1 - p

    @pl.when((l == 0) & (b == 0))
    def _init():
        hbuf[0] = jnp.dot(h_in[...], wemb[...],
                          preferred_element_type=jnp.float32) + bemb[...]
        xbuf[0] = x_in[...]
        vbuf[0] = jnp.zeros((n, 3), jnp.float32)

    h_full = hbuf[p]
    h_blk = hbuf[p, pl.ds(r0, bi), :]
    x_full = xbuf[p]
    x_blk = xbuf[p, pl.ds(r0, bi), :]
    v_blk = vbuf[p, pl.ds(r0, bi), :]

    xt3 = x_full.T
    dx0 = x_blk[:, 0:1] - xt3[0:1, :]
    dx1 = x_blk[:, 1:2] - xt3[1:2, :]
    dx2 = x_blk[:, 2:3] - xt3[2:3, :]
    d2 = (dx0 * dx0 + dx1 * dx1) + dx2 * dx2

    d2b = d2.astype(jnp.bfloat16).astype(jnp.float32)
    wcb = wc[l].astype(jnp.bfloat16).astype(jnp.float32)
    a_blk = (jnp.dot(h_blk, wa[l], preferred_element_type=jnp.float32)
             + be0[l])
    b_full = jnp.dot(h_full, wb[l], preferred_element_type=jnp.float32)
    pre = _silu_h(a_blk[:, None, :] + b_full[None, :, :]
                + d2b[:, :, None] * wcb[None, :, :])

    z = jnp.dot(pre.reshape(bi * n, hid), we1[l],
                preferred_element_type=jnp.float32) + be1[l]
    m = _silu_h(z).reshape(bi, n, hid)

    b_diag = jnp.dot(h_blk, wb[l], preferred_element_type=jnp.float32)
    pre_diag = _silu_h(a_blk + b_diag)
    m_diag = _silu_h(jnp.dot(pre_diag, we1[l],
                           preferred_element_type=jnp.float32) + be1[l])
    m_sum = jnp.sum(m, axis=1) - m_diag

    t = _silu_h(jnp.dot(m.reshape(bi * n, hid), wx0[l],
                      preferred_element_type=jnp.float32) + bx0[l])
    tb = t.astype(jnp.bfloat16).astype(jnp.float32)
    wx1b = wx1[l].astype(jnp.bfloat16).astype(jnp.float32)
    coef = (jnp.sum(tb.reshape(bi, n, hid) * wx1b[None, :, :], axis=-1)
            + bx1[l])

    scale = 1.0 / (n - 1)
    agg = jnp.concatenate(
        [jnp.sum(dx0 * coef, axis=1, keepdims=True),
         jnp.sum(dx1 * coef, axis=1, keepdims=True),
         jnp.sum(dx2 * coef, axis=1, keepdims=True)], axis=1) * scale

    hb16 = h_blk.astype(jnp.bfloat16).astype(jnp.float32)
    wv0b = wv0[l].astype(jnp.bfloat16).astype(jnp.float32)
    hv = jnp.sum(hb16 * wv0b, axis=1, keepdims=True) + bv0[l]
    v_new = hv * v_blk + agg
    x_new = x_blk + v_new

    u = _silu_h(jnp.dot(h_blk, wh0a[l], preferred_element_type=jnp.float32)
              + jnp.dot(m_sum, wh0b[l], preferred_element_type=jnp.float32)
              + bh0[l])
    h_new = h_blk + jnp.dot(u, wh1[l],
                            preferred_element_type=jnp.float32) + bh1[l]

    hbuf[q, pl.ds(r0, bi), :] = h_new
    xbuf[q, pl.ds(r0, bi), :] = x_new
    vbuf[q, pl.ds(r0, bi), :] = v_new

    @pl.when(l == depth - 1)
    def _emit():
        g = _silu_h(jnp.dot(h_new, wo0[...],
                          preferred_element_type=jnp.float32) + bo0[...])
        outh[pl.ds(r0, bi), :] = jnp.dot(
            g, wo1[...], preferred_element_type=jnp.float32) + bo1[...]
        outx[pl.ds(r0, bi), :] = x_new
        outv[pl.ds(r0, bi), :] = v_new


def kernel(h, x, params):
    n, in_f = h.shape
    layers = params["layers"]
    depth = len(layers)
    hid = layers[0]["e1"]["w"].shape[0]
    out_f = params["out1"]["w"].shape[1]
    bi = 128

    def stk(fn):
        return jnp.stack([fn(lp) for lp in layers])

    wa = stk(lambda lp: 0.5 * lp["e0"]["w"][:hid])
    wb = stk(lambda lp: 0.5 * lp["e0"]["w"][hid:2 * hid])
    wc = stk(lambda lp: 0.5 * lp["e0"]["w"][2 * hid][None])
    be0 = stk(lambda lp: 0.5 * lp["e0"]["b"][None])
    we1 = stk(lambda lp: 0.5 * lp["e1"]["w"])
    be1 = stk(lambda lp: 0.5 * lp["e1"]["b"][None])
    wx0 = stk(lambda lp: 0.5 * lp["x0"]["w"])
    bx0 = stk(lambda lp: 0.5 * lp["x0"]["b"][None])
    wx1 = stk(lambda lp: lp["x1"]["w"][:, 0][None])
    wh0a = stk(lambda lp: 0.5 * lp["h0"]["w"][:hid])
    wh0b = stk(lambda lp: 0.5 * lp["h0"]["w"][hid:])
    bh0 = stk(lambda lp: 0.5 * lp["h0"]["b"][None])
    wh1 = stk(lambda lp: lp["h1"]["w"])
    bh1 = stk(lambda lp: lp["h1"]["b"][None])
    wv0 = stk(lambda lp: lp["v0"]["w"][:, 0][None])
    bx1 = stk(lambda lp: lp["x1"]["b"][0])
    bv0 = stk(lambda lp: lp["v0"]["b"][0])

    wemb = params["emb_in"]["w"]
    bemb = params["emb_in"]["b"][None]
    wo0 = 0.5 * params["out0"]["w"]
    bo0 = 0.5 * params["out0"]["b"][None]
    wo1 = params["out1"]["w"]
    bo1 = params["out1"]["b"][None]

    f32 = jnp.float32
    vmem_full = lambda a: pl.BlockSpec(
        a.shape, lambda l, b, nd=a.ndim: (0,) * nd)
    smem_full = pl.BlockSpec(memory_space=pltpu.SMEM)

    args = (h, x, wa, wb, wc, be0, we1, be1, wx0, bx0, wx1,
            wh0a, wh0b, bh0, wh1, bh1, wv0, bx1, bv0,
            wemb, bemb, wo0, bo0, wo1, bo1)
    in_specs = []
    for name, a in zip(
            ("h", "x", "wa", "wb", "wc", "be0", "we1", "be1", "wx0", "bx0",
             "wx1", "wh0a", "wh0b", "bh0", "wh1", "bh1", "wv0", "bx1", "bv0",
             "wemb", "bemb", "wo0", "bo0", "wo1", "bo1"), args):
        in_specs.append(smem_full if name in ("bx1", "bv0") else vmem_full(a))

    out_shape = (
        jax.ShapeDtypeStruct((n, out_f), f32),
        jax.ShapeDtypeStruct((n, 3), f32),
        jax.ShapeDtypeStruct((n, 3), f32),
    )
    out_specs = (
        pl.BlockSpec((n, out_f), lambda l, b: (0, 0)),
        pl.BlockSpec((n, 3), lambda l, b: (0, 0)),
        pl.BlockSpec((n, 3), lambda l, b: (0, 0)),
    )

    body = functools.partial(_gnn_kernel, num_nodes=n, hid=hid, bi=bi,
                             depth=depth)
    return pl.pallas_call(
        body,
        grid=(depth, n // bi),
        in_specs=in_specs,
        out_specs=out_specs,
        out_shape=out_shape,
        scratch_shapes=[
            pltpu.VMEM((2, n, hid), f32),
            pltpu.VMEM((2, n, 3), f32),
            pltpu.VMEM((2, n, 3), f32),
        ],
        compiler_params=pltpu.CompilerParams(
            dimension_semantics=("arbitrary", "arbitrary")),
    )(*args)
